# baseline (speedup 1.0000x reference)
import jax
import jax.numpy as jnp
from jax import lax
from jax.experimental import pallas as pl
from jax.experimental.pallas import tpu as pltpu
from jax.experimental.pallas import tpu_sc as plsc

EMB_DIM = 50
EMB_PAD = 56
WINDOW = 5
HIDDEN = 512
OUT = 45
BATCH = 16384

NUM_WORKERS = 32
BATCH_PER_W = BATCH // NUM_WORKERS
CB = 16
NCHUNK = BATCH_PER_W // CB
GROWS = CB * WINDOW

KP = EMB_PAD * WINDOW
KT = 3
NBANDS = BATCH // 8

BM = 1024

_SPANS = []
for _w in range(WINDOW):
  for _s in (0, 16, 32, 40):
    _col = _w * EMB_PAD + _s
    _SPANS.append((_w, _s, _col // 128, _col % 128))
_ZSPANS = (24, 40, 56, 72, 88, 104, 112)


def _sc_gather_sum_body(wt, pt, iw, ip, isf, out,
                        xw, xp, xs, bufs, ob, sems, osems):
  wid = lax.axis_index("s") * 2 + lax.axis_index("c")
  band0 = wid * (BATCH_PER_W // 8)

  pltpu.sync_copy(iw.at[pl.ds(wid * NCHUNK, NCHUNK)], xw)
  pltpu.sync_copy(ip.at[pl.ds(wid * NCHUNK, NCHUNK)], xp)
  pltpu.sync_copy(isf.at[pl.ds(wid * NCHUNK, NCHUNK)], xs)

  for phase in range(2):
    for r in range(CB):
      z = jnp.zeros((16,), jnp.float32)
      for off in _ZSPANS:
        ob[phase, r // 8, 2, r % 8, pl.ds(off, 16)] = z

  def issue(j, phase):
    pltpu.async_copy(wt.at[xw.at[j]], bufs.at[phase, 0], sems.at[phase, 0])
    pltpu.async_copy(pt.at[xp.at[j]], bufs.at[phase, 1], sems.at[phase, 1])
    pltpu.async_copy(pt.at[xs.at[j]], bufs.at[phase, 2], sems.at[phase, 2])

  def drain(phase):
    for t in range(3):
      pltpu.make_async_copy(wt.at[xw.at[0]], bufs.at[phase, t],
                            sems.at[phase, t]).wait()

  def owait(phase):
    pltpu.make_async_copy(
        out.at[pl.ds(0, CB // 8)], ob.at[phase], osems.at[phase]).wait()

  def work(j, phase, first):
    drain(phase)
    nxt = j + 1
    nxt = jnp.where(nxt < NCHUNK, nxt, 0)
    issue(nxt, 1 - phase)
    if not first:
      owait(phase)

    for r in range(CB):
      band = r // 8
      r8 = r % 8
      g = r * WINDOW
      for w, s, ct, off in _SPANS:
        ob[phase, band, ct, r8, pl.ds(off, 16)] = (
            bufs[phase, 0, g + w, pl.ds(s, 16)]
            + bufs[phase, 1, g + w, pl.ds(s, 16)]
            + bufs[phase, 2, g + w, pl.ds(s, 16)])

    pltpu.async_copy(ob.at[phase],
                     out.at[pl.ds(band0 + j * (CB // 8), CB // 8)],
                     osems.at[phase])

  issue(0, 0)
  work(0, 0, True)
  work(1, 1, True)

  def pair(jj, carry):
    work(2 * jj, 0, False)
    work(2 * jj + 1, 1, False)
    return carry

  lax.fori_loop(1, NCHUNK // 2, pair, 0, unroll=False)
  drain(0)
  owait(0)
  owait(1)


_sc_gather_sum = pl.kernel(
    _sc_gather_sum_body,
    out_type=jax.ShapeDtypeStruct((NBANDS, KT, 8, 128), jnp.float32),
    mesh=plsc.VectorSubcoreMesh(core_axis_name="c", subcore_axis_name="s"),
    scratch_types=[
        pltpu.VMEM((NCHUNK, GROWS), jnp.int32),
        pltpu.VMEM((NCHUNK, GROWS), jnp.int32),
        pltpu.VMEM((NCHUNK, GROWS), jnp.int32),
        pltpu.VMEM((2, 3, GROWS, EMB_PAD), jnp.float32),
        pltpu.VMEM((2, CB // 8, KT, 8, 128), jnp.float32),
        pltpu.SemaphoreType.DMA((2, 3)),
        pltpu.SemaphoreType.DMA((2,)),
    ],
    compiler_params=pltpu.CompilerParams(use_tc_tiling_on_sc=False),
)


def _mlp_body(e_ref, w1_ref, b1_ref, w2_ref, b2_ref, o_ref):
  acc = b1_ref[...]
  for c in range(KT):
    xc = e_ref[:, c].reshape(BM, 128)
    acc = acc + jnp.dot(xc, w1_ref[c], preferred_element_type=jnp.float32)
  h = jnp.tanh(acc)
  lg = (jnp.dot(h, w2_ref[...], preferred_element_type=jnp.float32)
        + b2_ref[...])
  m = jnp.max(lg, axis=-1, keepdims=True)
  s = lg - m
  o_ref[...] = s - jnp.log(jnp.sum(jnp.exp(s), axis=-1, keepdims=True))


def _mlp(e4, w1p, b1, w2t, b2):
  return pl.pallas_call(
      _mlp_body,
      grid=(BATCH // BM,),
      in_specs=[
          pl.BlockSpec((BM // 8, KT, 8, 128), lambda i: (i, 0, 0, 0)),
          pl.BlockSpec((KT, 128, HIDDEN), lambda i: (0, 0, 0)),
          pl.BlockSpec((1, HIDDEN), lambda i: (0, 0)),
          pl.BlockSpec((HIDDEN, OUT), lambda i: (0, 0)),
          pl.BlockSpec((1, OUT), lambda i: (0, 0)),
      ],
      out_specs=pl.BlockSpec((BM, OUT), lambda i: (i, 0)),
      out_shape=jax.ShapeDtypeStruct((BATCH, OUT), jnp.float32),
  )(e4, w1p, b1, w2t, b2)


@jax.jit
def kernel(x, prefixes, suffixes, word_emb, prefix_emb, suffix_emb,
           W1, b1, W2, b2):
  del suffix_emb
  iw = x.astype(jnp.int32).reshape(-1, GROWS)
  ip = prefixes.astype(jnp.int32).reshape(-1, GROWS)
  isf = suffixes.astype(jnp.int32).reshape(-1, GROWS)
  wtp = jnp.pad(word_emb, ((0, 0), (0, EMB_PAD - EMB_DIM)))
  ptp = jnp.pad(prefix_emb, ((0, 0), (0, EMB_PAD - EMB_DIM)))
  e4 = _sc_gather_sum(wtp, ptp, iw, ip, isf)
  w1p = jnp.pad(W1.reshape(HIDDEN, WINDOW, EMB_DIM),
                ((0, 0), (0, 0), (0, EMB_PAD - EMB_DIM))).reshape(HIDDEN, KP)
  w1p = jnp.pad(w1p, ((0, 0), (0, KT * 128 - KP))).T.reshape(KT, 128, HIDDEN)
  return _mlp(e4, w1p, b1.reshape(1, HIDDEN), W2.T, b2.reshape(1, OUT))


# speedup vs baseline: 1.0964x; 1.0964x over previous
import jax
import jax.numpy as jnp
from jax import lax
from jax.experimental import pallas as pl
from jax.experimental.pallas import tpu as pltpu
from jax.experimental.pallas import tpu_sc as plsc

EMB_DIM = 50
EMB_PAD = 56
WINDOW = 5
HIDDEN = 512
OUT = 45
BATCH = 16384

NUM_WORKERS = 32
BATCH_PER_W = BATCH // NUM_WORKERS
CB = 16
NCHUNK = BATCH_PER_W // CB
GROWS = CB * WINDOW

KP = EMB_PAD * WINDOW
KT = 3
NBANDS = BATCH // 8

BM = 1024

_SPANS = []
for _w in range(WINDOW):
  for _s in (0, 16, 32, 40):
    _col = _w * EMB_PAD + _s
    _SPANS.append((_w, _s, _col // 128, _col % 128))
_ZSPANS = (24, 40, 56, 72, 88, 104, 112)


def _sc_gather_sum_body(wt, pt, iw, ip, isf, out,
                        xw, xp, xs, bufs, ob, sems, osems):
  wid = lax.axis_index("s") * 2 + lax.axis_index("c")
  band0 = wid * (BATCH_PER_W // 8)

  pltpu.sync_copy(iw.at[pl.ds(wid * NCHUNK, NCHUNK)], xw)
  pltpu.sync_copy(ip.at[pl.ds(wid * NCHUNK, NCHUNK)], xp)
  pltpu.sync_copy(isf.at[pl.ds(wid * NCHUNK, NCHUNK)], xs)

  def zrow(r, carry):
    z = jnp.zeros((16,), jnp.float32)
    for phase in range(2):
      for off in _ZSPANS:
        ob[phase, r // 8, 2, r % 8, pl.ds(off, 16)] = z
    return carry

  lax.fori_loop(0, CB, zrow, 0, unroll=False)

  def issue(j, phase):
    pltpu.async_copy(wt.at[xw.at[j]], bufs.at[phase, 0], sems.at[phase, 0])
    pltpu.async_copy(pt.at[xp.at[j]], bufs.at[phase, 1], sems.at[phase, 1])
    pltpu.async_copy(pt.at[xs.at[j]], bufs.at[phase, 2], sems.at[phase, 2])

  def drain(phase):
    for t in range(3):
      pltpu.make_async_copy(wt.at[xw.at[0]], bufs.at[phase, t],
                            sems.at[phase, t]).wait()

  def owait(phase):
    pltpu.make_async_copy(
        out.at[pl.ds(0, CB // 8)], ob.at[phase], osems.at[phase]).wait()

  def work(j, phase, first):
    drain(phase)
    nxt = j + 1
    nxt = jnp.where(nxt < NCHUNK, nxt, 0)
    issue(nxt, 1 - phase)
    if not first:
      owait(phase)

    def row(r, carry2):
      band = r // 8
      r8 = r % 8
      g = r * WINDOW
      for w, s, ct, off in _SPANS:
        ob[phase, band, ct, r8, pl.ds(off, 16)] = (
            bufs[phase, 0, g + w, pl.ds(s, 16)]
            + bufs[phase, 1, g + w, pl.ds(s, 16)]
            + bufs[phase, 2, g + w, pl.ds(s, 16)])
      return carry2

    lax.fori_loop(0, CB, row, 0, unroll=False)
    pltpu.async_copy(ob.at[phase],
                     out.at[pl.ds(band0 + j * (CB // 8), CB // 8)],
                     osems.at[phase])

  issue(0, 0)
  work(0, 0, True)
  work(1, 1, True)

  def pair(jj, carry):
    work(2 * jj, 0, False)
    work(2 * jj + 1, 1, False)
    return carry

  lax.fori_loop(1, NCHUNK // 2, pair, 0, unroll=False)
  drain(0)
  owait(0)
  owait(1)


_sc_gather_sum = pl.kernel(
    _sc_gather_sum_body,
    out_type=jax.ShapeDtypeStruct((NBANDS, KT, 8, 128), jnp.float32),
    mesh=plsc.VectorSubcoreMesh(core_axis_name="c", subcore_axis_name="s"),
    scratch_types=[
        pltpu.VMEM((NCHUNK, GROWS), jnp.int32),
        pltpu.VMEM((NCHUNK, GROWS), jnp.int32),
        pltpu.VMEM((NCHUNK, GROWS), jnp.int32),
        pltpu.VMEM((2, 3, GROWS, EMB_PAD), jnp.float32),
        pltpu.VMEM((2, CB // 8, KT, 8, 128), jnp.float32),
        pltpu.SemaphoreType.DMA((2, 3)),
        pltpu.SemaphoreType.DMA((2,)),
    ],
    compiler_params=pltpu.CompilerParams(use_tc_tiling_on_sc=False),
)


def _mlp_body(e_ref, w1_ref, b1_ref, w2_ref, b2_ref, o_ref):
  acc = b1_ref[...]
  for c in range(KT):
    xc = e_ref[:, c].reshape(BM, 128)
    acc = acc + jnp.dot(xc, w1_ref[c], preferred_element_type=jnp.float32)
  h = jnp.tanh(acc)
  lg = (jnp.dot(h, w2_ref[...], preferred_element_type=jnp.float32)
        + b2_ref[...])
  m = jnp.max(lg, axis=-1, keepdims=True)
  s = lg - m
  o_ref[...] = s - jnp.log(jnp.sum(jnp.exp(s), axis=-1, keepdims=True))


def _mlp(e4, w1p, b1, w2t, b2):
  return pl.pallas_call(
      _mlp_body,
      grid=(BATCH // BM,),
      in_specs=[
          pl.BlockSpec((BM // 8, KT, 8, 128), lambda i: (i, 0, 0, 0)),
          pl.BlockSpec((KT, 128, HIDDEN), lambda i: (0, 0, 0)),
          pl.BlockSpec((1, HIDDEN), lambda i: (0, 0)),
          pl.BlockSpec((HIDDEN, OUT), lambda i: (0, 0)),
          pl.BlockSpec((1, OUT), lambda i: (0, 0)),
      ],
      out_specs=pl.BlockSpec((BM, OUT), lambda i: (i, 0)),
      out_shape=jax.ShapeDtypeStruct((BATCH, OUT), jnp.float32),
  )(e4, w1p, b1, w2t, b2)


@jax.jit
def kernel(x, prefixes, suffixes, word_emb, prefix_emb, suffix_emb,
           W1, b1, W2, b2):
  del suffix_emb
  iw = x.astype(jnp.int32).reshape(-1, GROWS)
  ip = prefixes.astype(jnp.int32).reshape(-1, GROWS)
  isf = suffixes.astype(jnp.int32).reshape(-1, GROWS)
  wtp = jnp.pad(word_emb, ((0, 0), (0, EMB_PAD - EMB_DIM)))
  ptp = jnp.pad(prefix_emb, ((0, 0), (0, EMB_PAD - EMB_DIM)))
  e4 = _sc_gather_sum(wtp, ptp, iw, ip, isf)
  w1p = jnp.pad(W1.reshape(HIDDEN, WINDOW, EMB_DIM),
                ((0, 0), (0, 0), (0, EMB_PAD - EMB_DIM))).reshape(HIDDEN, KP)
  w1p = jnp.pad(w1p, ((0, 0), (0, KT * 128 - KP))).T.reshape(KT, 128, HIDDEN)
  return _mlp(e4, w1p, b1.reshape(1, HIDDEN), W2.T, b2.reshape(1, OUT))
